# baseline (speedup 1.0000x reference)
import functools

import jax
import jax.numpy as jnp
from jax import lax
from jax.experimental import pallas as pl
from jax.experimental.pallas import tpu as pltpu
from jax.experimental.pallas import tpu_sc as plsc

_INFO = plsc.get_sparse_core_info()
_NC = _INFO.num_cores
_NS = _INFO.num_subcores
_NW = _NC * _NS


@functools.cache
def _make_gather(B, D):
    b_per_w = B // _NW
    CH = 8
    NBUF = 14
    n_ch = b_per_w // CH
    mesh = plsc.VectorSubcoreMesh(core_axis_name="c", subcore_axis_name="s")

    @functools.partial(
        pl.kernel,
        mesh=mesh,
        out_type=jax.ShapeDtypeStruct((B, D), jnp.float32),
        scratch_types=[
            pltpu.VMEM((b_per_w,), jnp.int32),
            pltpu.VMEM((NBUF, CH, D), jnp.float32),
        ]
        + [pltpu.SemaphoreType.DMA] * (2 * NBUF),
    )
    def gather_kernel(idx_hbm, table_hbm, out_hbm, idx_v, bufs, *sems):
        gs, ss = sems[:NBUF], sems[NBUF:]
        wid = lax.axis_index("s") * _NC + lax.axis_index("c")
        base = wid * b_per_w
        pltpu.sync_copy(idx_hbm.at[pl.ds(base, b_per_w)], idx_v)

        def gather(j, b):
            return pltpu.async_copy(
                table_hbm.at[idx_v.at[pl.ds(j * CH, CH)]], bufs.at[b], gs[b]
            )

        def store(j, b):
            return pltpu.async_copy(
                bufs.at[b], out_hbm.at[pl.ds(base + j * CH, CH)], ss[b]
            )

        gh, sh = {}, {}
        for b in range(min(NBUF, n_ch)):
            gh[b] = gather(b, b)
        for j in range(n_ch):
            b = j % NBUF
            gh[b].wait()
            sh[b] = store(j, b)
            nxt = j + NBUF
            if nxt < n_ch:
                sh[b].wait()
                gh[b] = gather(nxt, b)
        for k in range(max(0, n_ch - NBUF), n_ch):
            sh[k % NBUF].wait()

    return gather_kernel


def kernel(seq_len, table):
    V, D = table.shape
    idx = jnp.arange(V, dtype=jnp.int32) % jnp.asarray(seq_len, jnp.int32)
    return _make_gather(V, D)(idx, table)


# speedup vs baseline: 1.0196x; 1.0196x over previous
import functools

import jax
import jax.numpy as jnp
from jax import lax
from jax.experimental import pallas as pl
from jax.experimental.pallas import tpu as pltpu
from jax.experimental.pallas import tpu_sc as plsc

_INFO = plsc.get_sparse_core_info()
_NC = _INFO.num_cores
_NS = _INFO.num_subcores
_NW = _NC * _NS


@functools.cache
def _make_gather(B, D):
    b_per_w = B // _NW
    CH = 16
    NBUF = 7
    n_ch = b_per_w // CH
    mesh = plsc.VectorSubcoreMesh(core_axis_name="c", subcore_axis_name="s")

    @functools.partial(
        pl.kernel,
        mesh=mesh,
        out_type=jax.ShapeDtypeStruct((B, D), jnp.float32),
        scratch_types=[
            pltpu.VMEM((b_per_w,), jnp.int32),
            pltpu.VMEM((NBUF, CH, D), jnp.float32),
        ]
        + [pltpu.SemaphoreType.DMA] * (2 * NBUF),
    )
    def gather_kernel(idx_hbm, table_hbm, out_hbm, idx_v, bufs, *sems):
        gs, ss = sems[:NBUF], sems[NBUF:]
        wid = lax.axis_index("s") * _NC + lax.axis_index("c")
        base = wid * b_per_w
        pltpu.sync_copy(idx_hbm.at[pl.ds(base, b_per_w)], idx_v)

        def gather(j, b):
            return pltpu.async_copy(
                table_hbm.at[idx_v.at[pl.ds(j * CH, CH)]], bufs.at[b], gs[b]
            )

        def store(j, b):
            return pltpu.async_copy(
                bufs.at[b], out_hbm.at[pl.ds(base + j * CH, CH)], ss[b]
            )

        gh, sh = {}, {}
        for b in range(min(NBUF, n_ch)):
            gh[b] = gather(b, b)
        for j in range(n_ch):
            b = j % NBUF
            gh[b].wait()
            sh[b] = store(j, b)
            nxt = j + NBUF
            if nxt < n_ch:
                sh[b].wait()
                gh[b] = gather(nxt, b)
        for k in range(max(0, n_ch - NBUF), n_ch):
            sh[k % NBUF].wait()

    return gather_kernel


def kernel(seq_len, table):
    V, D = table.shape
    idx = jnp.arange(V, dtype=jnp.int32) % jnp.asarray(seq_len, jnp.int32)
    return _make_gather(V, D)(idx, table)
